# baseline (speedup 1.0000x reference)
import jax
import jax.numpy as jnp
from jax.experimental import pallas as pl
from jax.experimental.pallas import tpu as pltpu
from jax.experimental.pallas import tpu_sc as plsc

_WINDOW = 800
_PCHUNK = 2048
_OCHUNK = 512


def _dotT(x, eye):
    return jax.lax.dot_general(
        x,
        eye,
        (((0,), (0,)), ((), ())),
        precision=jax.lax.Precision.HIGHEST,
        preferred_element_type=jnp.float32,
    )


def _pack_body(xa_ref, xb_ref, o_ref):
    eye = jnp.eye(xa_ref.shape[0], dtype=jnp.float32)
    at = jax.lax.dot_general(
        xa_ref[...], eye, (((0,), (0,)), ((), ())),
        precision=jax.lax.Precision.HIGHEST,
        preferred_element_type=jnp.float32,
    )
    bt = jax.lax.dot_general(
        xb_ref[...], eye, (((0,), (0,)), ((), ())),
        precision=jax.lax.Precision.HIGHEST,
        preferred_element_type=jnp.float32,
    )
    o_ref[...] = jnp.concatenate([at, bt], axis=1)


def _pack_table(wt, npairs):
    h, v = wt.shape
    maxb = -(-v // _PCHUNK) - 1
    return pl.pallas_call(
        _pack_body,
        grid=(npairs,),
        in_specs=[
            pl.BlockSpec((h, _PCHUNK), lambda i: (0, 2 * i)),
            pl.BlockSpec(
                (h, _PCHUNK), lambda i: (0, jnp.minimum(2 * i + 1, maxb))
            ),
        ],
        out_specs=pl.BlockSpec((_PCHUNK, 2 * h), lambda i: (i, 0)),
        out_shape=jax.ShapeDtypeStruct((npairs * _PCHUNK, 2 * h), wt.dtype),
    )(wt, wt)


def _make_untranspose_body(h, half_b):
    def body(x_ref, o_ref):
        x = x_ref[...]
        o_ref[0, :, :half_b] = x[:, :h].T
        o_ref[0, :, half_b:] = x[:, h:].T

    return body


def _rows_to_out(rows2, ll, b, h):
    hb = b // 2
    return pl.pallas_call(
        _make_untranspose_body(h, hb),
        grid=(ll,),
        in_specs=[pl.BlockSpec((hb, 2 * h), lambda l: (l, 0))],
        out_specs=pl.BlockSpec((1, h, b), lambda l: (l, 0, 0)),
        out_shape=jax.ShapeDtypeStruct((ll, h, b), rows2.dtype),
    )(rows2)


def _sc_gather(table, idx, n, h):
    mesh = plsc.VectorSubcoreMesh(
        core_axis_name="core", subcore_axis_name="subcore"
    )

    @pl.kernel(
        out_type=jax.ShapeDtypeStruct((n, h), table.dtype),
        mesh=mesh,
        compiler_params=pltpu.CompilerParams(use_tc_tiling_on_sc=False),
    )
    def run(table_hbm, idx_hbm, out_hbm):
        def body(i_vmem, o_vmem):
            pltpu.sync_copy(table_hbm.at[i_vmem.at[0]], o_vmem)

        pltpu.emit_pipeline(
            body,
            grid=(n // _WINDOW,),
            in_specs=[
                pl.BlockSpec((1, _WINDOW), index_map=lambda i: (0, i))
            ],
            out_specs=[
                pl.BlockSpec((_WINDOW, h), index_map=lambda i: (i, 0))
            ],
            core_axis_name=("core", "subcore"),
            dimension_semantics=(pltpu.PARALLEL,),
        )(idx_hbm, out_hbm)

    return run(table, idx)


def kernel(inputs, weight):
    b, ll = inputs.shape
    v, h = weight.shape
    n = b * ll
    npairs = -(-v // (2 * _PCHUNK))
    vpad = npairs * 2 * _PCHUNK

    wt = weight.T

    packed = _pack_table(wt, npairs)
    table_lin = packed.reshape(vpad, h)

    s = jnp.arange(b, dtype=jnp.int32)
    perm = (s % 2) * (b // 2) + s // 2
    idx_t = inputs.T.astype(jnp.int32)
    be = idx_t // _PCHUNK
    j_t = ((be // 2) * _PCHUNK + (idx_t % _PCHUNK)) * 2 + (be % 2)
    j = jnp.take(j_t, perm, axis=1).reshape(1, n)

    rows = _sc_gather(table_lin, j, n, h)
    out_t = _rows_to_out(rows.reshape(n // 2, 2 * h), ll, b, h)
    return jnp.transpose(out_t, (2, 0, 1))


# speedup vs baseline: 1.2666x; 1.2666x over previous
import jax
import jax.numpy as jnp
from jax.experimental import pallas as pl
from jax.experimental.pallas import tpu as pltpu
from jax.experimental.pallas import tpu_sc as plsc

_WINDOW = 800
_PCHUNK = 2048
_OCHUNK = 512


def _dotT(x, eye):
    return jax.lax.dot_general(
        x,
        eye,
        (((0,), (0,)), ((), ())),
        precision=jax.lax.Precision.HIGHEST,
        preferred_element_type=jnp.float32,
    )


def _pack_body(xa_ref, xb_ref, o_ref):
    o_ref[...] = jnp.concatenate([xa_ref[...].T, xb_ref[...].T], axis=1)


def _pack_table(wt, npairs):
    h, v = wt.shape
    maxb = -(-v // _PCHUNK) - 1
    return pl.pallas_call(
        _pack_body,
        grid=(npairs,),
        in_specs=[
            pl.BlockSpec((h, _PCHUNK), lambda i: (0, 2 * i)),
            pl.BlockSpec(
                (h, _PCHUNK), lambda i: (0, jnp.minimum(2 * i + 1, maxb))
            ),
        ],
        out_specs=pl.BlockSpec((_PCHUNK, 2 * h), lambda i: (i, 0)),
        out_shape=jax.ShapeDtypeStruct((npairs * _PCHUNK, 2 * h), wt.dtype),
    )(wt, wt)


def _make_untranspose_body(h, half_b):
    def body(x_ref, o_ref):
        x = x_ref[...]
        o_ref[0, :, :half_b] = x[:, :h].T
        o_ref[0, :, half_b:] = x[:, h:].T

    return body


def _rows_to_out(rows2, ll, b, h):
    hb = b // 2
    return pl.pallas_call(
        _make_untranspose_body(h, hb),
        grid=(ll,),
        in_specs=[pl.BlockSpec((hb, 2 * h), lambda l: (l, 0))],
        out_specs=pl.BlockSpec((1, h, b), lambda l: (l, 0, 0)),
        out_shape=jax.ShapeDtypeStruct((ll, h, b), rows2.dtype),
    )(rows2)


def _sc_gather(table, idx, n, h):
    mesh = plsc.VectorSubcoreMesh(
        core_axis_name="core", subcore_axis_name="subcore"
    )

    @pl.kernel(
        out_type=jax.ShapeDtypeStruct((n, h), table.dtype),
        mesh=mesh,
        compiler_params=pltpu.CompilerParams(use_tc_tiling_on_sc=False),
    )
    def run(table_hbm, idx_hbm, out_hbm):
        def body(i_vmem, o_vmem):
            pltpu.sync_copy(table_hbm.at[i_vmem.at[0]], o_vmem)

        pltpu.emit_pipeline(
            body,
            grid=(n // _WINDOW,),
            in_specs=[
                pl.BlockSpec((1, _WINDOW), index_map=lambda i: (0, i))
            ],
            out_specs=[
                pl.BlockSpec((_WINDOW, h), index_map=lambda i: (i, 0))
            ],
            core_axis_name=("core", "subcore"),
            dimension_semantics=(pltpu.PARALLEL,),
        )(idx_hbm, out_hbm)

    return run(table, idx)


def kernel(inputs, weight):
    b, ll = inputs.shape
    v, h = weight.shape
    n = b * ll
    npairs = -(-v // (2 * _PCHUNK))
    vpad = npairs * 2 * _PCHUNK

    wt = weight.T

    packed = _pack_table(wt, npairs)
    table_lin = packed.reshape(vpad, h)

    s = jnp.arange(b, dtype=jnp.int32)
    perm = (s % 2) * (b // 2) + s // 2
    idx_t = inputs.T.astype(jnp.int32)
    be = idx_t // _PCHUNK
    j_t = ((be // 2) * _PCHUNK + (idx_t % _PCHUNK)) * 2 + (be % 2)
    j = jnp.take(j_t, perm, axis=1).reshape(1, n)

    rows = _sc_gather(table_lin, j, n, h)
    out_t = _rows_to_out(rows.reshape(n // 2, 2 * h), ll, b, h)
    return jnp.transpose(out_t, (2, 0, 1))


# speedup vs baseline: 1.2694x; 1.0023x over previous
import jax
import jax.numpy as jnp
from jax.experimental import pallas as pl
from jax.experimental.pallas import tpu as pltpu
from jax.experimental.pallas import tpu_sc as plsc

_WINDOW = 800
_PCHUNK = 2048
_OCHUNK = 512


def _dotT(x, eye):
    return jax.lax.dot_general(
        x,
        eye,
        (((0,), (0,)), ((), ())),
        precision=jax.lax.Precision.HIGHEST,
        preferred_element_type=jnp.float32,
    )


def _pack_body(xa_ref, xb_ref, o_ref):
    o_ref[...] = jnp.concatenate([xa_ref[...].T, xb_ref[...].T], axis=1)


def _pack_table(wt, npairs):
    h, v = wt.shape
    maxb = -(-v // _PCHUNK) - 1
    return pl.pallas_call(
        _pack_body,
        grid=(npairs,),
        in_specs=[
            pl.BlockSpec((h, _PCHUNK), lambda i: (0, 2 * i)),
            pl.BlockSpec(
                (h, _PCHUNK), lambda i: (0, jnp.minimum(2 * i + 1, maxb))
            ),
        ],
        out_specs=pl.BlockSpec((_PCHUNK, 2 * h), lambda i: (i, 0)),
        out_shape=jax.ShapeDtypeStruct((npairs * _PCHUNK, 2 * h), wt.dtype),
        compiler_params=pltpu.CompilerParams(
            dimension_semantics=("parallel",)
        ),
    )(wt, wt)


def _make_untranspose_body(h, half_b):
    def body(x_ref, o_ref):
        x = x_ref[...]
        o_ref[0, :, :half_b] = x[:, :h].T
        o_ref[0, :, half_b:] = x[:, h:].T

    return body


def _rows_to_out(rows2, ll, b, h):
    hb = b // 2
    return pl.pallas_call(
        _make_untranspose_body(h, hb),
        grid=(ll,),
        in_specs=[pl.BlockSpec((hb, 2 * h), lambda l: (l, 0))],
        out_specs=pl.BlockSpec((1, h, b), lambda l: (l, 0, 0)),
        out_shape=jax.ShapeDtypeStruct((ll, h, b), rows2.dtype),
        compiler_params=pltpu.CompilerParams(
            dimension_semantics=("parallel",)
        ),
    )(rows2)


def _sc_gather(table, idx, n, h):
    mesh = plsc.VectorSubcoreMesh(
        core_axis_name="core", subcore_axis_name="subcore"
    )

    @pl.kernel(
        out_type=jax.ShapeDtypeStruct((n, h), table.dtype),
        mesh=mesh,
        compiler_params=pltpu.CompilerParams(use_tc_tiling_on_sc=False),
    )
    def run(table_hbm, idx_hbm, out_hbm):
        def body(i_vmem, o_vmem):
            pltpu.sync_copy(table_hbm.at[i_vmem.at[0]], o_vmem)

        pltpu.emit_pipeline(
            body,
            grid=(n // _WINDOW,),
            in_specs=[
                pl.BlockSpec((1, _WINDOW), index_map=lambda i: (0, i))
            ],
            out_specs=[
                pl.BlockSpec((_WINDOW, h), index_map=lambda i: (i, 0))
            ],
            core_axis_name=("core", "subcore"),
            dimension_semantics=(pltpu.PARALLEL,),
        )(idx_hbm, out_hbm)

    return run(table, idx)


def kernel(inputs, weight):
    b, ll = inputs.shape
    v, h = weight.shape
    n = b * ll
    npairs = -(-v // (2 * _PCHUNK))
    vpad = npairs * 2 * _PCHUNK

    wt = weight.T

    packed = _pack_table(wt, npairs)
    table_lin = packed.reshape(vpad, h)

    s = jnp.arange(b, dtype=jnp.int32)
    perm = (s % 2) * (b // 2) + s // 2
    idx_t = inputs.T.astype(jnp.int32)
    be = idx_t // _PCHUNK
    j_t = ((be // 2) * _PCHUNK + (idx_t % _PCHUNK)) * 2 + (be % 2)
    j = jnp.take(j_t, perm, axis=1).reshape(1, n)

    rows = _sc_gather(table_lin, j, n, h)
    out_t = _rows_to_out(rows.reshape(n // 2, 2 * h), ll, b, h)
    return jnp.transpose(out_t, (2, 0, 1))


# speedup vs baseline: 1.3085x; 1.0307x over previous
import jax
import jax.numpy as jnp
from jax.experimental import pallas as pl
from jax.experimental.pallas import tpu as pltpu
from jax.experimental.pallas import tpu_sc as plsc

_WINDOW = 800
_PCHUNK = 2048
_OCHUNK = 512


def _dotT(x, eye):
    return jax.lax.dot_general(
        x,
        eye,
        (((0,), (0,)), ((), ())),
        precision=jax.lax.Precision.HIGHEST,
        preferred_element_type=jnp.float32,
    )


def _pack_body(xa_ref, xb_ref, o_ref):
    o_ref[...] = jnp.concatenate([xa_ref[...].T, xb_ref[...].T], axis=1)


def _pack_table(wt, npairs):
    h, v = wt.shape
    maxb = -(-v // _PCHUNK) - 1
    return pl.pallas_call(
        _pack_body,
        grid=(npairs,),
        in_specs=[
            pl.BlockSpec((h, _PCHUNK), lambda i: (0, 2 * i)),
            pl.BlockSpec(
                (h, _PCHUNK), lambda i: (0, jnp.minimum(2 * i + 1, maxb))
            ),
        ],
        out_specs=pl.BlockSpec((_PCHUNK, 2 * h), lambda i: (i, 0)),
        out_shape=jax.ShapeDtypeStruct((npairs * _PCHUNK, 2 * h), wt.dtype),
        compiler_params=pltpu.CompilerParams(
            dimension_semantics=("parallel",)
        ),
    )(wt, wt)


def _make_untranspose_body(h, half_b, with_prev):
    def body(*refs):
        x_ref, o_ref = refs[0], refs[-1]
        x = x_ref[...]
        o_ref[0, :, :half_b] = x[:, :h].T
        o_ref[0, :, half_b:] = x[:, h:].T

    return body


def _rows_to_out(rows2, prev, ll, b, h, l_off):
    hb = b // 2
    grid_l = (2 * rows2.shape[0]) // b
    in_specs = [pl.BlockSpec((hb, 2 * h), lambda l: (l, 0))]
    operands = [rows2]
    aliases = {}
    if prev is not None:
        in_specs.append(pl.BlockSpec(memory_space=pltpu.MemorySpace.HBM))
        operands.append(prev)
        aliases = {1: 0}
    return pl.pallas_call(
        _make_untranspose_body(h, hb, prev is not None),
        grid=(grid_l,),
        in_specs=in_specs,
        out_specs=pl.BlockSpec((1, h, b), lambda l: (l + l_off, 0, 0)),
        out_shape=jax.ShapeDtypeStruct((ll, h, b), rows2.dtype),
        input_output_aliases=aliases,
        compiler_params=pltpu.CompilerParams(
            dimension_semantics=("arbitrary",)
        ),
    )(*operands)


def _sc_gather(table, idx, n, h):
    mesh = plsc.VectorSubcoreMesh(
        core_axis_name="core", subcore_axis_name="subcore"
    )

    @pl.kernel(
        out_type=jax.ShapeDtypeStruct((n, h), table.dtype),
        mesh=mesh,
        compiler_params=pltpu.CompilerParams(use_tc_tiling_on_sc=False),
    )
    def run(table_hbm, idx_hbm, out_hbm):
        def body(i_vmem, o_vmem):
            pltpu.sync_copy(table_hbm.at[i_vmem.at[0]], o_vmem)

        pltpu.emit_pipeline(
            body,
            grid=(n // _WINDOW,),
            in_specs=[
                pl.BlockSpec((1, _WINDOW), index_map=lambda i: (0, i))
            ],
            out_specs=[
                pl.BlockSpec((_WINDOW, h), index_map=lambda i: (i, 0))
            ],
            core_axis_name=("core", "subcore"),
            dimension_semantics=(pltpu.PARALLEL,),
        )(idx_hbm, out_hbm)

    return run(table, idx)


def kernel(inputs, weight):
    b, ll = inputs.shape
    v, h = weight.shape
    n = b * ll
    npairs = -(-v // (2 * _PCHUNK))
    vpad = npairs * 2 * _PCHUNK

    wt = weight.T

    packed = _pack_table(wt, npairs)
    table_lin = packed.reshape(vpad, h)

    s = jnp.arange(b, dtype=jnp.int32)
    perm = (s % 2) * (b // 2) + s // 2
    idx_t = inputs.T.astype(jnp.int32)
    be = idx_t // _PCHUNK
    j_t = ((be // 2) * _PCHUNK + (idx_t % _PCHUNK)) * 2 + (be % 2)
    j = jnp.take(j_t, perm, axis=1).reshape(1, n)

    nh = n // 2
    rows_a = _sc_gather(table_lin, j[:, :nh], nh, h)
    rows_b = _sc_gather(table_lin, j[:, nh:], nh, h)
    out1 = _rows_to_out(rows_a.reshape(nh // 2, 2 * h), None, ll, b, h, 0)
    out_t = _rows_to_out(
        rows_b.reshape(nh // 2, 2 * h), out1, ll, b, h, ll // 2
    )
    return jnp.transpose(out_t, (2, 0, 1))


# speedup vs baseline: 1.3393x; 1.0236x over previous
import jax
import jax.numpy as jnp
from jax.experimental import pallas as pl
from jax.experimental.pallas import tpu as pltpu
from jax.experimental.pallas import tpu_sc as plsc

_WINDOW = 800
_PCHUNK = 2048
_OCHUNK = 512


def _dotT(x, eye):
    return jax.lax.dot_general(
        x,
        eye,
        (((0,), (0,)), ((), ())),
        precision=jax.lax.Precision.HIGHEST,
        preferred_element_type=jnp.float32,
    )


def _pack_body(xa_ref, xb_ref, o_ref):
    o_ref[...] = jnp.concatenate([xa_ref[...].T, xb_ref[...].T], axis=1)


def _pack_table(wt, npairs):
    h, v = wt.shape
    maxb = -(-v // _PCHUNK) - 1
    return pl.pallas_call(
        _pack_body,
        grid=(npairs,),
        in_specs=[
            pl.BlockSpec((h, _PCHUNK), lambda i: (0, 2 * i)),
            pl.BlockSpec(
                (h, _PCHUNK), lambda i: (0, jnp.minimum(2 * i + 1, maxb))
            ),
        ],
        out_specs=pl.BlockSpec((_PCHUNK, 2 * h), lambda i: (i, 0)),
        out_shape=jax.ShapeDtypeStruct((npairs * _PCHUNK, 2 * h), wt.dtype),
        compiler_params=pltpu.CompilerParams(
            dimension_semantics=("parallel",)
        ),
    )(wt, wt)


def _make_untranspose_body(h, half_b, with_prev):
    def body(*refs):
        x_ref, o_ref = refs[0], refs[-1]
        x = x_ref[...]
        o_ref[0, :, :half_b] = x[:, :h].T
        o_ref[0, :, half_b:] = x[:, h:].T

    return body


def _rows_to_out(rows2, prev, ll, b, h, l_off):
    hb = b // 2
    grid_l = (2 * rows2.shape[0]) // b
    in_specs = [pl.BlockSpec((hb, 2 * h), lambda l: (l, 0))]
    operands = [rows2]
    aliases = {}
    if prev is not None:
        in_specs.append(pl.BlockSpec(memory_space=pltpu.MemorySpace.HBM))
        operands.append(prev)
        aliases = {1: 0}
    return pl.pallas_call(
        _make_untranspose_body(h, hb, prev is not None),
        grid=(grid_l,),
        in_specs=in_specs,
        out_specs=pl.BlockSpec((1, h, b), lambda l: (l + l_off, 0, 0)),
        out_shape=jax.ShapeDtypeStruct((ll, h, b), rows2.dtype),
        input_output_aliases=aliases,
        compiler_params=pltpu.CompilerParams(
            dimension_semantics=("arbitrary",)
        ),
    )(*operands)


def _sc_gather(table, idx, n, h):
    mesh = plsc.VectorSubcoreMesh(
        core_axis_name="core", subcore_axis_name="subcore"
    )

    @pl.kernel(
        out_type=jax.ShapeDtypeStruct((n, h), table.dtype),
        mesh=mesh,
        compiler_params=pltpu.CompilerParams(use_tc_tiling_on_sc=False),
    )
    def run(table_hbm, idx_hbm, out_hbm):
        def body(i_vmem, o_vmem):
            pltpu.sync_copy(table_hbm.at[i_vmem.at[0]], o_vmem)

        pltpu.emit_pipeline(
            body,
            grid=(n // _WINDOW,),
            in_specs=[
                pl.BlockSpec((1, _WINDOW), index_map=lambda i: (0, i))
            ],
            out_specs=[
                pl.BlockSpec((_WINDOW, h), index_map=lambda i: (i, 0))
            ],
            core_axis_name=("core", "subcore"),
            dimension_semantics=(pltpu.PARALLEL,),
        )(idx_hbm, out_hbm)

    return run(table, idx)


def kernel(inputs, weight):
    b, ll = inputs.shape
    v, h = weight.shape
    n = b * ll
    npairs = -(-v // (2 * _PCHUNK))
    vpad = npairs * 2 * _PCHUNK

    wt = weight.T

    packed = _pack_table(wt, npairs)
    table_lin = packed.reshape(vpad, h)

    s = jnp.arange(b, dtype=jnp.int32)
    perm = (s % 2) * (b // 2) + s // 2
    idx_t = inputs.T.astype(jnp.int32)
    be = idx_t // _PCHUNK
    j_t = ((be // 2) * _PCHUNK + (idx_t % _PCHUNK)) * 2 + (be % 2)
    j = jnp.take(j_t, perm, axis=1).reshape(1, n)

    k = 4
    nh = n // k
    lh = ll // k
    out_t = None
    for i in range(k):
        rows_i = _sc_gather(table_lin, j[:, i * nh : (i + 1) * nh], nh, h)
        out_t = _rows_to_out(
            rows_i.reshape(nh // 2, 2 * h), out_t, ll, b, h, i * lh
        )
    return jnp.transpose(out_t, (2, 0, 1))


# speedup vs baseline: 1.3473x; 1.0060x over previous
import jax
import jax.numpy as jnp
from jax.experimental import pallas as pl
from jax.experimental.pallas import tpu as pltpu
from jax.experimental.pallas import tpu_sc as plsc

_WINDOW = 800
_PCHUNK = 2048
_OCHUNK = 512


def _dotT(x, eye):
    return jax.lax.dot_general(
        x,
        eye,
        (((0,), (0,)), ((), ())),
        precision=jax.lax.Precision.HIGHEST,
        preferred_element_type=jnp.float32,
    )


def _pack_body(xa_ref, xb_ref, o_ref):
    o_ref[...] = jnp.concatenate([xa_ref[...].T, xb_ref[...].T], axis=1)


def _pack_table(wt, npairs):
    h, v = wt.shape
    maxb = -(-v // _PCHUNK) - 1
    return pl.pallas_call(
        _pack_body,
        grid=(npairs,),
        in_specs=[
            pl.BlockSpec((h, _PCHUNK), lambda i: (0, 2 * i)),
            pl.BlockSpec(
                (h, _PCHUNK), lambda i: (0, jnp.minimum(2 * i + 1, maxb))
            ),
        ],
        out_specs=pl.BlockSpec((_PCHUNK, 2 * h), lambda i: (i, 0)),
        out_shape=jax.ShapeDtypeStruct((npairs * _PCHUNK, 2 * h), wt.dtype),
        compiler_params=pltpu.CompilerParams(
            dimension_semantics=("parallel",)
        ),
    )(wt, wt)


def _make_untranspose_body(h, half_b, with_prev):
    def body(*refs):
        x_ref, o_ref = refs[0], refs[-1]
        x = x_ref[...]
        o_ref[0, :, :half_b] = x[:, :h].T
        o_ref[0, :, half_b:] = x[:, h:].T

    return body


def _rows_to_out(rows2, prev, ll, b, h, l_off):
    hb = b // 2
    grid_l = (2 * rows2.shape[0]) // b
    in_specs = [pl.BlockSpec((hb, 2 * h), lambda l: (l, 0))]
    operands = [rows2]
    aliases = {}
    if prev is not None:
        in_specs.append(pl.BlockSpec(memory_space=pltpu.MemorySpace.HBM))
        operands.append(prev)
        aliases = {1: 0}
    return pl.pallas_call(
        _make_untranspose_body(h, hb, prev is not None),
        grid=(grid_l,),
        in_specs=in_specs,
        out_specs=pl.BlockSpec((1, h, b), lambda l: (l + l_off, 0, 0)),
        out_shape=jax.ShapeDtypeStruct((ll, h, b), rows2.dtype),
        input_output_aliases=aliases,
        compiler_params=pltpu.CompilerParams(
            dimension_semantics=("arbitrary",)
        ),
    )(*operands)


def _sc_gather(table, idx, n, h):
    mesh = plsc.VectorSubcoreMesh(
        core_axis_name="core", subcore_axis_name="subcore"
    )

    @pl.kernel(
        out_type=jax.ShapeDtypeStruct((n, h), table.dtype),
        mesh=mesh,
        compiler_params=pltpu.CompilerParams(use_tc_tiling_on_sc=False),
    )
    def run(table_hbm, idx_hbm, out_hbm):
        def body(i_vmem, o_vmem):
            pltpu.sync_copy(table_hbm.at[i_vmem.at[0]], o_vmem)

        pltpu.emit_pipeline(
            body,
            grid=(n // _WINDOW,),
            in_specs=[
                pl.BlockSpec((1, _WINDOW), index_map=lambda i: (0, i))
            ],
            out_specs=[
                pl.BlockSpec((_WINDOW, h), index_map=lambda i: (i, 0))
            ],
            core_axis_name=("core", "subcore"),
            dimension_semantics=(pltpu.PARALLEL,),
        )(idx_hbm, out_hbm)

    return run(table, idx)


def kernel(inputs, weight):
    b, ll = inputs.shape
    v, h = weight.shape
    n = b * ll
    npairs = -(-v // (2 * _PCHUNK))
    vpad = npairs * 2 * _PCHUNK

    wt = weight.T

    packed = _pack_table(wt, npairs)
    table_lin = packed.reshape(vpad, h)

    s = jnp.arange(b, dtype=jnp.int32)
    perm = (s % 2) * (b // 2) + s // 2
    idx_t = inputs.T.astype(jnp.int32)
    be = idx_t // _PCHUNK
    j_t = ((be // 2) * _PCHUNK + (idx_t % _PCHUNK)) * 2 + (be % 2)
    j = jnp.take(j_t, perm, axis=1).reshape(1, n)

    k = 8
    nh = n // k
    lh = ll // k
    out_t = None
    for i in range(k):
        rows_i = _sc_gather(table_lin, j[:, i * nh : (i + 1) * nh], nh, h)
        out_t = _rows_to_out(
            rows_i.reshape(nh // 2, 2 * h), out_t, ll, b, h, i * lh
        )
    return jnp.transpose(out_t, (2, 0, 1))


# speedup vs baseline: 1.3483x; 1.0007x over previous
import jax
import jax.numpy as jnp
from jax.experimental import pallas as pl
from jax.experimental.pallas import tpu as pltpu
from jax.experimental.pallas import tpu_sc as plsc

_WINDOW = 800
_PCHUNK = 2048


def _pack_body(xa_ref, xb_ref, o_ref):
    o_ref[...] = jnp.concatenate([xa_ref[...].T, xb_ref[...].T], axis=1)


def _pack_table(wt, npairs):
    h, v = wt.shape
    maxb = -(-v // _PCHUNK) - 1
    return pl.pallas_call(
        _pack_body,
        grid=(npairs,),
        in_specs=[
            pl.BlockSpec((h, _PCHUNK), lambda i: (0, 2 * i)),
            pl.BlockSpec(
                (h, _PCHUNK), lambda i: (0, jnp.minimum(2 * i + 1, maxb))
            ),
        ],
        out_specs=pl.BlockSpec((_PCHUNK, 2 * h), lambda i: (i, 0)),
        out_shape=jax.ShapeDtypeStruct((npairs * _PCHUNK, 2 * h), wt.dtype),
        compiler_params=pltpu.CompilerParams(
            dimension_semantics=("parallel",)
        ),
    )(wt, wt)


def _make_untranspose_body(h, half_b):
    def body(*refs):
        x_ref, o_ref = refs[0], refs[-1]
        x = x_ref[...]
        o_ref[0, :, :half_b] = x[:, :h].T
        o_ref[0, :, half_b:] = x[:, h:].T

    return body


def _rows_to_out(rows2, prev, ll, b, h, l_off):
    hb = b // 2
    grid_l = (2 * rows2.shape[0]) // b
    in_specs = [pl.BlockSpec((hb, 2 * h), lambda l: (l, 0))]
    operands = [rows2]
    aliases = {}
    if prev is not None:
        in_specs.append(pl.BlockSpec(memory_space=pltpu.MemorySpace.HBM))
        operands.append(prev)
        aliases = {1: 0}
    return pl.pallas_call(
        _make_untranspose_body(h, hb),
        grid=(grid_l,),
        in_specs=in_specs,
        out_specs=pl.BlockSpec((1, h, b), lambda l: (l + l_off, 0, 0)),
        out_shape=jax.ShapeDtypeStruct((ll, h, b), rows2.dtype),
        input_output_aliases=aliases,
        compiler_params=pltpu.CompilerParams(
            dimension_semantics=("arbitrary",)
        ),
    )(*operands)


def _sc_gather(table, idx, n, h):
    mesh = plsc.VectorSubcoreMesh(
        core_axis_name="core", subcore_axis_name="subcore"
    )

    @pl.kernel(
        out_type=jax.ShapeDtypeStruct((n, h), table.dtype),
        mesh=mesh,
        compiler_params=pltpu.CompilerParams(use_tc_tiling_on_sc=False),
    )
    def run(table_hbm, idx_hbm, out_hbm):
        def body(i_vmem, o_vmem):
            pltpu.sync_copy(table_hbm.at[i_vmem.at[0]], o_vmem)

        pltpu.emit_pipeline(
            body,
            grid=(n // _WINDOW,),
            in_specs=[
                pl.BlockSpec((1, _WINDOW), index_map=lambda i: (0, i))
            ],
            out_specs=[
                pl.BlockSpec((_WINDOW, h), index_map=lambda i: (i, 0))
            ],
            core_axis_name=("core", "subcore"),
            dimension_semantics=(pltpu.PARALLEL,),
        )(idx_hbm, out_hbm)

    return run(table, idx)


def kernel(inputs, weight):
    b, ll = inputs.shape
    v, h = weight.shape
    n = b * ll
    npairs = -(-v // (2 * _PCHUNK))
    vpad = npairs * 2 * _PCHUNK

    wt = weight.T

    packed = _pack_table(wt, npairs)
    table_lin = packed.reshape(vpad, h)

    s = jnp.arange(b, dtype=jnp.int32)
    perm = (s % 2) * (b // 2) + s // 2
    idx_t = inputs.T.astype(jnp.int32)
    be = idx_t // _PCHUNK
    j_t = ((be // 2) * _PCHUNK + (idx_t % _PCHUNK)) * 2 + (be % 2)
    j = jnp.take(j_t, perm, axis=1).reshape(1, n)

    k = 8
    nh = n // k
    lh = ll // k
    out_t = None
    for i in range(k):
        rows_i = _sc_gather(table_lin, j[:, i * nh : (i + 1) * nh], nh, h)
        out_t = _rows_to_out(
            rows_i.reshape(nh // 2, 2 * h), out_t, ll, b, h, i * lh
        )
    return jnp.transpose(out_t, (2, 0, 1))
